# baseline (speedup 1.0000x reference)
import functools
import jax
import jax.numpy as jnp
from jax import lax
from jax.experimental import pallas as pl
from jax.experimental.pallas import tpu as pltpu

H = 2048
NH = 16
NKV = 4
HD = 128
ROT = 64
E = 16
TOPK = 2
FF = 1024
EPS = 1e-05
RSF = 1.0
T = 2048

QKV = NH * HD + 2 * NKV * HD

BLK = 128
NASSIGN = T * TOPK
P = NASSIGN + E * BLK
NBLOCKS = P // BLK


def _qkv_body(x_ref, w_ref, ln_ref, o_ref):
    x = x_ref[...]
    v = jnp.mean(jnp.square(x), axis=-1, keepdims=True)
    h = x * lax.rsqrt(v + EPS) * ln_ref[...]
    o_ref[...] = jnp.dot(h, w_ref[...].T, preferred_element_type=jnp.float32)


def _qkv_proj(x, qkv_w, ln_w):
    BT, BN = 256, 512
    return pl.pallas_call(
        _qkv_body,
        grid=(T // BT, QKV // BN),
        in_specs=[
            pl.BlockSpec((BT, H), lambda i, j: (i, 0)),
            pl.BlockSpec((BN, H), lambda i, j: (j, 0)),
            pl.BlockSpec((1, H), lambda i, j: (0, 0)),
        ],
        out_specs=pl.BlockSpec((BT, BN), lambda i, j: (i, j)),
        out_shape=jax.ShapeDtypeStruct((T, QKV), jnp.float32),
    )(x, qkv_w, ln_w.reshape(1, H))


def _rope(x, pos):
    inv = 1.0 / (10000.0 ** (
        lax.broadcasted_iota(jnp.int32, (1, ROT // 2), 1).astype(jnp.float32)
        * 2.0 / ROT))
    ang = pos * inv
    cos = jnp.cos(ang)
    sin = jnp.sin(ang)
    x1 = x[:, : ROT // 2]
    x2 = x[:, ROT // 2: ROT]
    xp = x[:, ROT:]
    return jnp.concatenate([x1 * cos - x2 * sin, x2 * cos + x1 * sin, xp],
                           axis=-1)


def _rms(x, w):
    v = jnp.mean(jnp.square(x), axis=-1, keepdims=True)
    return x * lax.rsqrt(v + EPS) * w


def _attn_body(qkv_ref, pos_ref, qn_ref, kn_ref, o_ref):
    h = pl.program_id(0)
    qb = pl.program_id(1)
    BQ = o_ref.shape[0]
    g = h // (NH // NKV)
    q = qkv_ref[pl.ds(qb * BQ, BQ), pl.ds(h * HD, HD)]
    k = qkv_ref[:, pl.ds(NH * HD + g * HD, HD)]
    v = qkv_ref[:, pl.ds(NH * HD + NKV * HD + g * HD, HD)]
    qpos = pos_ref[0, pl.ds(qb * BQ, BQ)].reshape(BQ, 1)
    kpos = pos_ref[0, :].reshape(T, 1)
    q = _rope(_rms(q, qn_ref[...]), qpos)
    k = _rope(_rms(k, kn_ref[...]), kpos)
    s = jnp.dot(q, k.T, preferred_element_type=jnp.float32) * (HD ** -0.5)
    rows = qb * BQ + lax.broadcasted_iota(jnp.int32, (BQ, T), 0)
    cols = lax.broadcasted_iota(jnp.int32, (BQ, T), 1)
    s = jnp.where(cols <= rows, s, -1e30)
    m = jnp.max(s, axis=-1, keepdims=True)
    p = jnp.exp(s - m)
    p = p / jnp.sum(p, axis=-1, keepdims=True)
    o_ref[...] = jnp.dot(p, v, preferred_element_type=jnp.float32)


def _attention(qkv, positions, qn, kn):
    BQ = 512
    return pl.pallas_call(
        _attn_body,
        grid=(NH, T // BQ),
        in_specs=[
            pl.BlockSpec((T, QKV), lambda h, i: (0, 0)),
            pl.BlockSpec((1, T), lambda h, i: (0, 0)),
            pl.BlockSpec((1, HD), lambda h, i: (0, 0)),
            pl.BlockSpec((1, HD), lambda h, i: (0, 0)),
        ],
        out_specs=pl.BlockSpec((BQ, HD), lambda h, i: (i, h)),
        out_shape=jax.ShapeDtypeStruct((T, NH * HD), jnp.float32),
    )(qkv, positions.astype(jnp.float32).reshape(1, T),
      qn.reshape(1, HD), kn.reshape(1, HD))


def _oproj_body(a_ref, w_ref, res_ref, ln_ref, gw_ref,
                r2_ref, h2_ref, lg_ref, acc):
    kk = pl.program_id(1)
    nk = pl.num_programs(1)

    @pl.when(kk == 0)
    def _():
        acc[...] = jnp.zeros_like(acc)

    acc[...] += jnp.dot(a_ref[...], w_ref[...].T,
                        preferred_element_type=jnp.float32)

    @pl.when(kk == nk - 1)
    def _():
        r2 = acc[...] + res_ref[...]
        r2_ref[...] = r2
        v = jnp.mean(jnp.square(r2), axis=-1, keepdims=True)
        h2 = r2 * lax.rsqrt(v + EPS) * ln_ref[...]
        h2_ref[...] = h2
        lg_ref[...] = jnp.dot(h2, gw_ref[...].T,
                              preferred_element_type=jnp.float32)


def _oproj(attn, o_w, residual, post_ln_w, gate_w_pad):
    BT, BK = 256, 512
    return pl.pallas_call(
        _oproj_body,
        grid=(T // BT, H // BK),
        in_specs=[
            pl.BlockSpec((BT, BK), lambda i, k: (i, k)),
            pl.BlockSpec((H, BK), lambda i, k: (0, k)),
            pl.BlockSpec((BT, H), lambda i, k: (i, 0)),
            pl.BlockSpec((1, H), lambda i, k: (0, 0)),
            pl.BlockSpec((128, H), lambda i, k: (0, 0)),
        ],
        out_specs=[
            pl.BlockSpec((BT, H), lambda i, k: (i, 0)),
            pl.BlockSpec((BT, H), lambda i, k: (i, 0)),
            pl.BlockSpec((BT, 128), lambda i, k: (i, 0)),
        ],
        out_shape=[
            jax.ShapeDtypeStruct((T, H), jnp.float32),
            jax.ShapeDtypeStruct((T, H), jnp.float32),
            jax.ShapeDtypeStruct((T, 128), jnp.float32),
        ],
        scratch_shapes=[pltpu.VMEM((BT, H), jnp.float32)],
    )(attn, o_w, residual, post_ln_w.reshape(1, H), gate_w_pad)


def _cumsum0(y):
    n = y.shape[0]
    k = 1
    while k < n:
        y = y + jnp.concatenate(
            [jnp.zeros((k, y.shape[1]), y.dtype), y[:-k]], axis=0)
        k *= 2
    return y


def _plan_body(lg_ref, bias_ref, dest_ref, wflat_ref, gid_ref):
    lg = lg_ref[...]
    cols = lax.broadcasted_iota(jnp.int32, (T, 128), 1)
    valid = cols < E
    s = jnp.where(valid, 1.0 / (1.0 + jnp.exp(-lg)), 0.0)
    sc = jnp.where(valid, s + bias_ref[...], -1e30)

    m1 = jnp.max(sc, axis=-1, keepdims=True)
    i1 = jnp.min(jnp.where(sc >= m1, cols, E), axis=-1, keepdims=True)
    sc2 = jnp.where(cols == i1, -1e30, sc)
    m2 = jnp.max(sc2, axis=-1, keepdims=True)
    i2 = jnp.min(jnp.where(sc2 >= m2, cols, E), axis=-1, keepdims=True)

    w1 = jnp.sum(jnp.where(cols == i1, s, 0.0), axis=-1, keepdims=True)
    w2 = jnp.sum(jnp.where(cols == i2, s, 0.0), axis=-1, keepdims=True)
    den = w1 + w2 + 1e-20
    w1 = w1 / den
    w2 = w2 / den

    oh1 = (cols == i1).astype(jnp.float32)
    oh2 = (cols == i2).astype(jnp.float32)
    c1x = _cumsum0(oh1) - oh1
    c2x = _cumsum0(oh2) - oh2
    rank1 = jnp.sum((c1x + c2x) * oh1, axis=-1, keepdims=True)
    rank2 = jnp.sum((c1x + c2x + oh1) * oh2, axis=-1, keepdims=True)
    counts = jnp.sum(oh1 + oh2, axis=0, keepdims=True)

    nblk = jnp.ceil(counts / BLK)
    tri = (lax.broadcasted_iota(jnp.int32, (128, 128), 0)
           <= lax.broadcasted_iota(jnp.int32, (128, 128), 1)
           ).astype(jnp.float32)
    cblk = jnp.dot(nblk, tri, preferred_element_type=jnp.float32)
    start_blk = cblk - nblk
    aligned = start_blk * BLK
    dest1 = jnp.sum(aligned * oh1, axis=-1, keepdims=True) + rank1
    dest2 = jnp.sum(aligned * oh2, axis=-1, keepdims=True) + rank2

    is0 = (cols == 0).astype(jnp.float32)
    is1 = (cols == 1).astype(jnp.float32)
    dest_ref[...] = (dest1 * is0 + dest2 * is1).astype(jnp.int32)
    wflat_ref[...] = w1 * is0 + w2 * is1

    brow = lax.broadcasted_iota(jnp.int32, (128, 128), 0).astype(jnp.float32)
    used = lax.broadcasted_iota(jnp.int32, (128, 128), 1) < E
    ge = jnp.where(used, (brow >= cblk).astype(jnp.int32), 0)
    gid = jnp.minimum(jnp.sum(ge, axis=-1, keepdims=True), E - 1)
    gid_ref[...] = jnp.broadcast_to(gid, (128, 128)).astype(jnp.int32)


def _router_plan(logits, e_bias_pad):
    return pl.pallas_call(
        _plan_body,
        grid=(1,),
        in_specs=[
            pl.BlockSpec((T, 128), lambda i: (0, 0)),
            pl.BlockSpec((1, 128), lambda i: (0, 0)),
        ],
        out_specs=[
            pl.BlockSpec((T, 128), lambda i: (0, 0)),
            pl.BlockSpec((T, 128), lambda i: (0, 0)),
            pl.BlockSpec((128, 128), lambda i: (0, 0)),
        ],
        out_shape=[
            jax.ShapeDtypeStruct((T, 128), jnp.int32),
            jax.ShapeDtypeStruct((T, 128), jnp.float32),
            jax.ShapeDtypeStruct((128, 128), jnp.int32),
        ],
    )(logits, e_bias_pad)


def _gffn_body(gid_ref, xg_ref, wg_ref, wu_ref, wd_ref, o_ref, acc):
    f = pl.program_id(1)
    nf = pl.num_programs(1)

    @pl.when(f == 0)
    def _():
        acc[...] = jnp.zeros_like(acc)

    x = xg_ref[...]
    g = jnp.dot(x, wg_ref[0].T, preferred_element_type=jnp.float32)
    u = jnp.dot(x, wu_ref[0].T, preferred_element_type=jnp.float32)
    act = (g / (1.0 + jnp.exp(-g))) * u
    acc[...] += jnp.dot(act, wd_ref[0],
                        preferred_element_type=jnp.float32)

    @pl.when(f == nf - 1)
    def _():
        o_ref[...] = acc[...]


def _grouped_ffn(xg, w_gate, w_up, w_down, gid):
    FT = 512
    grid_spec = pltpu.PrefetchScalarGridSpec(
        num_scalar_prefetch=1,
        grid=(NBLOCKS, FF // FT),
        in_specs=[
            pl.BlockSpec((BLK, H), lambda b, f, gid: (b, 0)),
            pl.BlockSpec((1, FT, H), lambda b, f, gid: (gid[b], f, 0)),
            pl.BlockSpec((1, FT, H), lambda b, f, gid: (gid[b], f, 0)),
            pl.BlockSpec((1, FT, H), lambda b, f, gid: (gid[b], f, 0)),
        ],
        out_specs=pl.BlockSpec((BLK, H), lambda b, f, gid: (b, 0)),
        scratch_shapes=[pltpu.VMEM((BLK, H), jnp.float32)],
    )
    return pl.pallas_call(
        _gffn_body,
        grid_spec=grid_spec,
        out_shape=jax.ShapeDtypeStruct((P, H), jnp.float32),
    )(gid, xg, w_gate, w_up, w_down)


def _shared_body(x_ref, wg_ref, wu_ref, wd_ref, base_ref, o_ref, acc):
    f = pl.program_id(1)
    nf = pl.num_programs(1)

    @pl.when(f == 0)
    def _():
        acc[...] = jnp.zeros_like(acc)

    x = x_ref[...]
    g = jnp.dot(x, wg_ref[...].T, preferred_element_type=jnp.float32)
    u = jnp.dot(x, wu_ref[...].T, preferred_element_type=jnp.float32)
    act = (g / (1.0 + jnp.exp(-g))) * u
    acc[...] += jnp.dot(act, wd_ref[...].T,
                        preferred_element_type=jnp.float32)

    @pl.when(f == nf - 1)
    def _():
        o_ref[...] = acc[...] + base_ref[...]


def _shared_ffn(h2, sh_gate, sh_up, sh_down, base):
    BT, FT = 256, 512
    return pl.pallas_call(
        _shared_body,
        grid=(T // BT, FF // FT),
        in_specs=[
            pl.BlockSpec((BT, H), lambda i, f: (i, 0)),
            pl.BlockSpec((FT, H), lambda i, f: (f, 0)),
            pl.BlockSpec((FT, H), lambda i, f: (f, 0)),
            pl.BlockSpec((H, FT), lambda i, f: (0, f)),
            pl.BlockSpec((BT, H), lambda i, f: (i, 0)),
        ],
        out_specs=pl.BlockSpec((BT, H), lambda i, f: (i, 0)),
        out_shape=jax.ShapeDtypeStruct((T, H), jnp.float32),
        scratch_shapes=[pltpu.VMEM((BT, H), jnp.float32)],
    )(h2, sh_gate, sh_up, sh_down, base)


def kernel(positions, hidden_states, input_ln_w, qkv_w, q_norm_w, k_norm_w,
           o_w, post_ln_w, gate_w, e_bias, w_gate_up, w_down,
           sh_gate_up, sh_down):
    x = hidden_states
    qkv = _qkv_proj(x, qkv_w, input_ln_w)
    attn = _attention(qkv, positions, q_norm_w, k_norm_w)

    gate_w_pad = jnp.zeros((128, H), jnp.float32).at[:E].set(gate_w)
    r2, h2, logits = _oproj(attn, o_w, x, post_ln_w, gate_w_pad)

    bias_pad = jnp.full((1, 128), -1e30, jnp.float32).at[0, :E].set(e_bias)
    dest2d, wflat2d, gid2d = _router_plan(logits, bias_pad)
    pos2 = dest2d[:, :TOPK]
    w2 = wflat2d[:, :TOPK]
    dest = pos2.reshape(NASSIGN)
    gid = gid2d[:NBLOCKS, 0]

    tok = jnp.arange(NASSIGN, dtype=jnp.int32) // TOPK
    xg = jnp.zeros((P, H), jnp.float32).at[dest].set(h2[tok])

    w_gate = w_gate_up[:, :FF, :]
    w_up = w_gate_up[:, FF:, :]
    out_pad = _grouped_ffn(xg, w_gate, w_up, w_down, gid)

    sh_gate = sh_gate_up[:FF, :]
    sh_up = sh_gate_up[FF:, :]
    base = _shared_ffn(h2, sh_gate, sh_up, sh_down, r2)

    routed = (out_pad[pos2[:, 0]] * w2[:, 0:1]
              + out_pad[pos2[:, 1]] * w2[:, 1:2])
    return base + routed * RSF


# speedup vs baseline: 1.2000x; 1.2000x over previous
import functools
import jax
import jax.numpy as jnp
from jax import lax
from jax.experimental import pallas as pl
from jax.experimental.pallas import tpu as pltpu

H = 2048
NH = 16
NKV = 4
HD = 128
ROT = 64
E = 16
TOPK = 2
FF = 1024
EPS = 1e-05
RSF = 1.0
T = 2048

QKV = NH * HD + 2 * NKV * HD

BLK = 128
NASSIGN = T * TOPK
P = NASSIGN + E * BLK
NBLOCKS = P // BLK


def _qkv_body(x_ref, w_ref, ln_ref, o_ref):
    x = x_ref[...]
    v = jnp.mean(jnp.square(x), axis=-1, keepdims=True)
    h = x * lax.rsqrt(v + EPS) * ln_ref[...]
    o_ref[...] = jnp.dot(h, w_ref[...].T, preferred_element_type=jnp.float32)


def _qkv_proj(x, qkv_w, ln_w):
    BT, BN = 256, 512
    return pl.pallas_call(
        _qkv_body,
        grid=(T // BT, QKV // BN),
        in_specs=[
            pl.BlockSpec((BT, H), lambda i, j: (i, 0)),
            pl.BlockSpec((BN, H), lambda i, j: (j, 0)),
            pl.BlockSpec((1, H), lambda i, j: (0, 0)),
        ],
        out_specs=pl.BlockSpec((BT, BN), lambda i, j: (i, j)),
        out_shape=jax.ShapeDtypeStruct((T, QKV), jnp.float32),
    )(x, qkv_w, ln_w.reshape(1, H))


def _rope(x, pos):
    inv = 1.0 / (10000.0 ** (
        lax.broadcasted_iota(jnp.int32, (1, ROT // 2), 1).astype(jnp.float32)
        * 2.0 / ROT))
    ang = pos * inv
    cos = jnp.cos(ang)
    sin = jnp.sin(ang)
    x1 = x[:, : ROT // 2]
    x2 = x[:, ROT // 2: ROT]
    xp = x[:, ROT:]
    return jnp.concatenate([x1 * cos - x2 * sin, x2 * cos + x1 * sin, xp],
                           axis=-1)


def _rms(x, w):
    v = jnp.mean(jnp.square(x), axis=-1, keepdims=True)
    return x * lax.rsqrt(v + EPS) * w


BQA = 512


def _attn_body(qkv_ref, pos_ref, qn_ref, kn_ref, o_ref, acc, mrow, lrow):
    h = pl.program_id(0)
    qb = pl.program_id(1)
    g = h // (NH // NKV)
    q = qkv_ref[pl.ds(qb * BQA, BQA), pl.ds(h * HD, HD)]
    qpos = pos_ref[0, pl.ds(qb * BQA, BQA)].reshape(BQA, 1)
    q = _rope(_rms(q, qn_ref[...]), qpos) * (HD ** -0.5)

    acc[...] = jnp.zeros_like(acc)
    mrow[...] = jnp.full_like(mrow, -1e30)
    lrow[...] = jnp.zeros_like(lrow)

    def kv(j):
        k = qkv_ref[pl.ds(j * BQA, BQA), pl.ds(NH * HD + g * HD, HD)]
        v = qkv_ref[pl.ds(j * BQA, BQA),
                    pl.ds(NH * HD + NKV * HD + g * HD, HD)]
        kpos = pos_ref[0, pl.ds(j * BQA, BQA)].reshape(BQA, 1)
        k = _rope(_rms(k, kn_ref[...]), kpos)
        return k, v

    def step(s, v):
        m_prev = mrow[:, :1]
        m_new = jnp.maximum(m_prev, jnp.max(s, axis=-1, keepdims=True))
        corr = jnp.exp(m_prev - m_new)
        p = jnp.exp(s - m_new)
        lrow[...] = lrow[...] * corr + jnp.sum(p, axis=-1, keepdims=True)
        mrow[...] = jnp.broadcast_to(m_new, mrow.shape)
        acc[...] = acc[...] * corr + jnp.dot(
            p, v, preferred_element_type=jnp.float32)

    def body(j, _):
        k, v = kv(j)
        step(jnp.dot(q, k.T, preferred_element_type=jnp.float32), v)
        return 0

    lax.fori_loop(0, qb, body, 0)

    k, v = kv(qb)
    s = jnp.dot(q, k.T, preferred_element_type=jnp.float32)
    rows = lax.broadcasted_iota(jnp.int32, (BQA, BQA), 0)
    cols = lax.broadcasted_iota(jnp.int32, (BQA, BQA), 1)
    s = jnp.where(cols <= rows, s, -1e30)
    step(s, v)

    o_ref[...] = acc[...] / lrow[:, :1]


def _attention(qkv, positions, qn, kn):
    return pl.pallas_call(
        _attn_body,
        grid=(NH, T // BQA),
        in_specs=[
            pl.BlockSpec((T, QKV), lambda h, i: (0, 0)),
            pl.BlockSpec((1, T), lambda h, i: (0, 0)),
            pl.BlockSpec((1, HD), lambda h, i: (0, 0)),
            pl.BlockSpec((1, HD), lambda h, i: (0, 0)),
        ],
        out_specs=pl.BlockSpec((BQA, HD), lambda h, i: (i, h)),
        out_shape=jax.ShapeDtypeStruct((T, NH * HD), jnp.float32),
        scratch_shapes=[
            pltpu.VMEM((BQA, HD), jnp.float32),
            pltpu.VMEM((BQA, 128), jnp.float32),
            pltpu.VMEM((BQA, 128), jnp.float32),
        ],
    )(qkv, positions.astype(jnp.float32).reshape(1, T),
      qn.reshape(1, HD), kn.reshape(1, HD))


def _oproj_body(a_ref, w_ref, res_ref, ln_ref, gw_ref,
                r2_ref, h2_ref, lg_ref, acc):
    kk = pl.program_id(1)
    nk = pl.num_programs(1)

    @pl.when(kk == 0)
    def _():
        acc[...] = jnp.zeros_like(acc)

    acc[...] += jnp.dot(a_ref[...], w_ref[...].T,
                        preferred_element_type=jnp.float32)

    @pl.when(kk == nk - 1)
    def _():
        r2 = acc[...] + res_ref[...]
        r2_ref[...] = r2
        v = jnp.mean(jnp.square(r2), axis=-1, keepdims=True)
        h2 = r2 * lax.rsqrt(v + EPS) * ln_ref[...]
        h2_ref[...] = h2
        lg_ref[...] = jnp.dot(h2, gw_ref[...].T,
                              preferred_element_type=jnp.float32)


def _oproj(attn, o_w, residual, post_ln_w, gate_w_pad):
    BT, BK = 256, 512
    return pl.pallas_call(
        _oproj_body,
        grid=(T // BT, H // BK),
        in_specs=[
            pl.BlockSpec((BT, BK), lambda i, k: (i, k)),
            pl.BlockSpec((H, BK), lambda i, k: (0, k)),
            pl.BlockSpec((BT, H), lambda i, k: (i, 0)),
            pl.BlockSpec((1, H), lambda i, k: (0, 0)),
            pl.BlockSpec((128, H), lambda i, k: (0, 0)),
        ],
        out_specs=[
            pl.BlockSpec((BT, H), lambda i, k: (i, 0)),
            pl.BlockSpec((BT, H), lambda i, k: (i, 0)),
            pl.BlockSpec((BT, 128), lambda i, k: (i, 0)),
        ],
        out_shape=[
            jax.ShapeDtypeStruct((T, H), jnp.float32),
            jax.ShapeDtypeStruct((T, H), jnp.float32),
            jax.ShapeDtypeStruct((T, 128), jnp.float32),
        ],
        scratch_shapes=[pltpu.VMEM((BT, H), jnp.float32)],
    )(attn, o_w, residual, post_ln_w.reshape(1, H), gate_w_pad)


def _cumsum0(y):
    n = y.shape[0]
    k = 1
    while k < n:
        y = y + jnp.concatenate(
            [jnp.zeros((k, y.shape[1]), y.dtype), y[:-k]], axis=0)
        k *= 2
    return y


def _plan_body(lg_ref, bias_ref, dest_ref, wflat_ref, gid_ref):
    lg = lg_ref[...]
    cols = lax.broadcasted_iota(jnp.int32, (T, 128), 1)
    valid = cols < E
    s = jnp.where(valid, 1.0 / (1.0 + jnp.exp(-lg)), 0.0)
    sc = jnp.where(valid, s + bias_ref[...], -1e30)

    m1 = jnp.max(sc, axis=-1, keepdims=True)
    i1 = jnp.min(jnp.where(sc >= m1, cols, E), axis=-1, keepdims=True)
    sc2 = jnp.where(cols == i1, -1e30, sc)
    m2 = jnp.max(sc2, axis=-1, keepdims=True)
    i2 = jnp.min(jnp.where(sc2 >= m2, cols, E), axis=-1, keepdims=True)

    w1 = jnp.sum(jnp.where(cols == i1, s, 0.0), axis=-1, keepdims=True)
    w2 = jnp.sum(jnp.where(cols == i2, s, 0.0), axis=-1, keepdims=True)
    den = w1 + w2 + 1e-20
    w1 = w1 / den
    w2 = w2 / den

    oh1 = (cols == i1).astype(jnp.float32)
    oh2 = (cols == i2).astype(jnp.float32)
    c1x = _cumsum0(oh1) - oh1
    c2x = _cumsum0(oh2) - oh2
    rank1 = jnp.sum((c1x + c2x) * oh1, axis=-1, keepdims=True)
    rank2 = jnp.sum((c1x + c2x + oh1) * oh2, axis=-1, keepdims=True)
    counts = jnp.sum(oh1 + oh2, axis=0, keepdims=True)

    nblk = jnp.ceil(counts / BLK)
    tri = (lax.broadcasted_iota(jnp.int32, (128, 128), 0)
           <= lax.broadcasted_iota(jnp.int32, (128, 128), 1)
           ).astype(jnp.float32)
    cblk = jnp.dot(nblk, tri, preferred_element_type=jnp.float32)
    start_blk = cblk - nblk
    aligned = start_blk * BLK
    dest1 = jnp.sum(aligned * oh1, axis=-1, keepdims=True) + rank1
    dest2 = jnp.sum(aligned * oh2, axis=-1, keepdims=True) + rank2

    is0 = (cols == 0).astype(jnp.float32)
    is1 = (cols == 1).astype(jnp.float32)
    dest_ref[...] = (dest1 * is0 + dest2 * is1).astype(jnp.int32)
    wflat_ref[...] = w1 * is0 + w2 * is1

    brow = lax.broadcasted_iota(jnp.int32, (128, 128), 0).astype(jnp.float32)
    used = lax.broadcasted_iota(jnp.int32, (128, 128), 1) < E
    ge = jnp.where(used, (brow >= cblk).astype(jnp.int32), 0)
    gid = jnp.minimum(jnp.sum(ge, axis=-1, keepdims=True), E - 1)
    gid_ref[...] = jnp.broadcast_to(gid, (128, 128)).astype(jnp.int32)


def _router_plan(logits, e_bias_pad):
    return pl.pallas_call(
        _plan_body,
        grid=(1,),
        in_specs=[
            pl.BlockSpec((T, 128), lambda i: (0, 0)),
            pl.BlockSpec((1, 128), lambda i: (0, 0)),
        ],
        out_specs=[
            pl.BlockSpec((T, 128), lambda i: (0, 0)),
            pl.BlockSpec((T, 128), lambda i: (0, 0)),
            pl.BlockSpec((128, 128), lambda i: (0, 0)),
        ],
        out_shape=[
            jax.ShapeDtypeStruct((T, 128), jnp.int32),
            jax.ShapeDtypeStruct((T, 128), jnp.float32),
            jax.ShapeDtypeStruct((128, 128), jnp.int32),
        ],
    )(logits, e_bias_pad)


def _gffn_body(gid_ref, xg_ref, wg_ref, wu_ref, wd_ref, o_ref):
    x = xg_ref[...]
    g = jnp.dot(x, wg_ref[0].T, preferred_element_type=jnp.float32)
    u = jnp.dot(x, wu_ref[0].T, preferred_element_type=jnp.float32)
    act = (g / (1.0 + jnp.exp(-g))) * u
    o_ref[...] = jnp.dot(act, wd_ref[0], preferred_element_type=jnp.float32)


def _grouped_ffn(xg, w_gate, w_up, w_down, gid):
    grid_spec = pltpu.PrefetchScalarGridSpec(
        num_scalar_prefetch=1,
        grid=(NBLOCKS,),
        in_specs=[
            pl.BlockSpec((BLK, H), lambda b, gid: (b, 0)),
            pl.BlockSpec((1, FF, H), lambda b, gid: (gid[b], 0, 0)),
            pl.BlockSpec((1, FF, H), lambda b, gid: (gid[b], 0, 0)),
            pl.BlockSpec((1, FF, H), lambda b, gid: (gid[b], 0, 0)),
        ],
        out_specs=pl.BlockSpec((BLK, H), lambda b, gid: (b, 0)),
    )
    return pl.pallas_call(
        _gffn_body,
        grid_spec=grid_spec,
        out_shape=jax.ShapeDtypeStruct((P, H), jnp.float32),
    )(gid, xg, w_gate, w_up, w_down)


def _shared_body(x_ref, wg_ref, wu_ref, wd_ref, base_ref, o_ref, acc):
    f = pl.program_id(1)
    nf = pl.num_programs(1)

    @pl.when(f == 0)
    def _():
        acc[...] = jnp.zeros_like(acc)

    x = x_ref[...]
    g = jnp.dot(x, wg_ref[...].T, preferred_element_type=jnp.float32)
    u = jnp.dot(x, wu_ref[...].T, preferred_element_type=jnp.float32)
    act = (g / (1.0 + jnp.exp(-g))) * u
    acc[...] += jnp.dot(act, wd_ref[...].T,
                        preferred_element_type=jnp.float32)

    @pl.when(f == nf - 1)
    def _():
        o_ref[...] = acc[...] + base_ref[...]


def _shared_ffn(h2, sh_gate, sh_up, sh_down, base):
    BT, FT = 256, 512
    return pl.pallas_call(
        _shared_body,
        grid=(T // BT, FF // FT),
        in_specs=[
            pl.BlockSpec((BT, H), lambda i, f: (i, 0)),
            pl.BlockSpec((FT, H), lambda i, f: (f, 0)),
            pl.BlockSpec((FT, H), lambda i, f: (f, 0)),
            pl.BlockSpec((H, FT), lambda i, f: (0, f)),
            pl.BlockSpec((BT, H), lambda i, f: (i, 0)),
        ],
        out_specs=pl.BlockSpec((BT, H), lambda i, f: (i, 0)),
        out_shape=jax.ShapeDtypeStruct((T, H), jnp.float32),
        scratch_shapes=[pltpu.VMEM((BT, H), jnp.float32)],
    )(h2, sh_gate, sh_up, sh_down, base)


def kernel(positions, hidden_states, input_ln_w, qkv_w, q_norm_w, k_norm_w,
           o_w, post_ln_w, gate_w, e_bias, w_gate_up, w_down,
           sh_gate_up, sh_down):
    x = hidden_states
    qkv = _qkv_proj(x, qkv_w, input_ln_w)
    attn = _attention(qkv, positions, q_norm_w, k_norm_w)

    gate_w_pad = jnp.zeros((128, H), jnp.float32).at[:E].set(gate_w)
    r2, h2, logits = _oproj(attn, o_w, x, post_ln_w, gate_w_pad)

    bias_pad = jnp.full((1, 128), -1e30, jnp.float32).at[0, :E].set(e_bias)
    dest2d, wflat2d, gid2d = _router_plan(logits, bias_pad)
    pos2 = dest2d[:, :TOPK]
    w2 = wflat2d[:, :TOPK]
    dest = pos2.reshape(NASSIGN)
    gid = gid2d[:NBLOCKS, 0]

    tok = jnp.arange(NASSIGN, dtype=jnp.int32) // TOPK
    xg = jnp.zeros((P, H), jnp.float32).at[dest].set(h2[tok])

    w_gate = w_gate_up[:, :FF, :]
    w_up = w_gate_up[:, FF:, :]
    out_pad = _grouped_ffn(xg, w_gate, w_up, w_down, gid)

    sh_gate = sh_gate_up[:FF, :]
    sh_up = sh_gate_up[FF:, :]
    base = _shared_ffn(h2, sh_gate, sh_up, sh_down, r2)

    routed = (out_pad[pos2[:, 0]] * w2[:, 0:1]
              + out_pad[pos2[:, 1]] * w2[:, 1:2])
    return base + routed * RSF


# speedup vs baseline: 1.3404x; 1.1171x over previous
import functools
import jax
import jax.numpy as jnp
from jax import lax
from jax.experimental import pallas as pl
from jax.experimental.pallas import tpu as pltpu

H = 2048
NH = 16
NKV = 4
HD = 128
ROT = 64
E = 16
TOPK = 2
FF = 1024
EPS = 1e-05
RSF = 1.0
T = 2048

QKV = NH * HD + 2 * NKV * HD

BLK = 128
NASSIGN = T * TOPK
P = NASSIGN + E * BLK
NBLOCKS = P // BLK


def _qkv_body(x_ref, w_ref, ln_ref, o_ref):
    x = x_ref[...]
    v = jnp.mean(jnp.square(x), axis=-1, keepdims=True)
    h = x * lax.rsqrt(v + EPS) * ln_ref[...]
    o_ref[...] = jnp.dot(h, w_ref[...].T, preferred_element_type=jnp.float32)


def _qkv_proj(x, qkv_w, ln_w):
    BT, BN = 256, 512
    return pl.pallas_call(
        _qkv_body,
        grid=(T // BT, QKV // BN),
        in_specs=[
            pl.BlockSpec((BT, H), lambda i, j: (i, 0)),
            pl.BlockSpec((BN, H), lambda i, j: (j, 0)),
            pl.BlockSpec((1, H), lambda i, j: (0, 0)),
        ],
        out_specs=pl.BlockSpec((BT, BN), lambda i, j: (i, j)),
        out_shape=jax.ShapeDtypeStruct((T, QKV), jnp.float32),
    )(x, qkv_w, ln_w.reshape(1, H))


def _rope(x, cos, sin):
    x1 = x[:, : ROT // 2]
    x2 = x[:, ROT // 2: ROT]
    xp = x[:, ROT:]
    return jnp.concatenate([x1 * cos - x2 * sin, x2 * cos + x1 * sin, xp],
                           axis=-1)


def _rms(x, w):
    v = jnp.mean(jnp.square(x), axis=-1, keepdims=True)
    return x * lax.rsqrt(v + EPS) * w


BQA = 512


def _attn_body(qkv_ref, cos_ref, sin_ref, qn_ref, kn_ref, o_ref, smat):
    h = pl.program_id(0)
    qb = pl.program_id(1)
    g = h // (NH // NKV)
    q = qkv_ref[pl.ds(qb * BQA, BQA), pl.ds(h * HD, HD)]
    qcos = cos_ref[pl.ds(qb * BQA, BQA), :]
    qsin = sin_ref[pl.ds(qb * BQA, BQA), :]
    q = _rope(_rms(q, qn_ref[...]), qcos, qsin)

    for j in range(T // BQA):
        @pl.when(j <= qb)
        def _():
            k = qkv_ref[pl.ds(j * BQA, BQA), pl.ds(NH * HD + g * HD, HD)]
            k = _rope(_rms(k, kn_ref[...]),
                      cos_ref[pl.ds(j * BQA, BQA), :],
                      sin_ref[pl.ds(j * BQA, BQA), :])
            s = jnp.dot(q, k.T, preferred_element_type=jnp.float32)
            s = s * (HD ** -0.5)
            @pl.when(j == qb)
            def _():
                rows = lax.broadcasted_iota(jnp.int32, (BQA, BQA), 0)
                cols = lax.broadcasted_iota(jnp.int32, (BQA, BQA), 1)
                smat[:, pl.ds(j * BQA, BQA)] = jnp.where(
                    cols <= rows, s, -1e30)
            @pl.when(j < qb)
            def _():
                smat[:, pl.ds(j * BQA, BQA)] = s
        @pl.when(j > qb)
        def _():
            smat[:, pl.ds(j * BQA, BQA)] = jnp.full((BQA, BQA), -1e30,
                                                    jnp.float32)

    s = smat[...]
    m = jnp.max(s, axis=-1, keepdims=True)
    p = jnp.exp(s - m)
    p = p / jnp.sum(p, axis=-1, keepdims=True)
    v = qkv_ref[:, pl.ds(NH * HD + NKV * HD + g * HD, HD)]
    o_ref[...] = jnp.dot(p, v, preferred_element_type=jnp.float32)


def _attention(qkv, cos, sin, qn, kn):
    return pl.pallas_call(
        _attn_body,
        grid=(NH, T // BQA),
        in_specs=[
            pl.BlockSpec((T, QKV), lambda h, i: (0, 0)),
            pl.BlockSpec((T, ROT // 2), lambda h, i: (0, 0)),
            pl.BlockSpec((T, ROT // 2), lambda h, i: (0, 0)),
            pl.BlockSpec((1, HD), lambda h, i: (0, 0)),
            pl.BlockSpec((1, HD), lambda h, i: (0, 0)),
        ],
        out_specs=pl.BlockSpec((BQA, HD), lambda h, i: (i, h)),
        out_shape=jax.ShapeDtypeStruct((T, NH * HD), jnp.float32),
        scratch_shapes=[
            pltpu.VMEM((BQA, T), jnp.float32),
        ],
    )(qkv, cos, sin, qn.reshape(1, HD), kn.reshape(1, HD))


def _oproj_body(a_ref, w_ref, res_ref, ln_ref, gw_ref,
                r2_ref, h2_ref, lg_ref, acc):
    kk = pl.program_id(1)
    nk = pl.num_programs(1)

    @pl.when(kk == 0)
    def _():
        acc[...] = jnp.zeros_like(acc)

    acc[...] += jnp.dot(a_ref[...], w_ref[...].T,
                        preferred_element_type=jnp.float32)

    @pl.when(kk == nk - 1)
    def _():
        r2 = acc[...] + res_ref[...]
        r2_ref[...] = r2
        v = jnp.mean(jnp.square(r2), axis=-1, keepdims=True)
        h2 = r2 * lax.rsqrt(v + EPS) * ln_ref[...]
        h2_ref[...] = h2
        lg_ref[...] = jnp.dot(h2, gw_ref[...].T,
                              preferred_element_type=jnp.float32)


def _oproj(attn, o_w, residual, post_ln_w, gate_w_pad):
    BT, BK = 256, 512
    return pl.pallas_call(
        _oproj_body,
        grid=(T // BT, H // BK),
        in_specs=[
            pl.BlockSpec((BT, BK), lambda i, k: (i, k)),
            pl.BlockSpec((H, BK), lambda i, k: (0, k)),
            pl.BlockSpec((BT, H), lambda i, k: (i, 0)),
            pl.BlockSpec((1, H), lambda i, k: (0, 0)),
            pl.BlockSpec((128, H), lambda i, k: (0, 0)),
        ],
        out_specs=[
            pl.BlockSpec((BT, H), lambda i, k: (i, 0)),
            pl.BlockSpec((BT, H), lambda i, k: (i, 0)),
            pl.BlockSpec((BT, 128), lambda i, k: (i, 0)),
        ],
        out_shape=[
            jax.ShapeDtypeStruct((T, H), jnp.float32),
            jax.ShapeDtypeStruct((T, H), jnp.float32),
            jax.ShapeDtypeStruct((T, 128), jnp.float32),
        ],
        scratch_shapes=[pltpu.VMEM((BT, H), jnp.float32)],
    )(attn, o_w, residual, post_ln_w.reshape(1, H), gate_w_pad)


def _cumsum0(y):
    n = y.shape[0]
    k = 1
    while k < n:
        y = y + jnp.concatenate(
            [jnp.zeros((k, y.shape[1]), y.dtype), y[:-k]], axis=0)
        k *= 2
    return y


def _plan_body(lg_ref, bias_ref, dest_ref, wflat_ref, gid_ref):
    lg = lg_ref[...]
    cols = lax.broadcasted_iota(jnp.int32, (T, 128), 1)
    valid = cols < E
    s = jnp.where(valid, 1.0 / (1.0 + jnp.exp(-lg)), 0.0)
    sc = jnp.where(valid, s + bias_ref[...], -1e30)

    m1 = jnp.max(sc, axis=-1, keepdims=True)
    i1 = jnp.min(jnp.where(sc >= m1, cols, E), axis=-1, keepdims=True)
    sc2 = jnp.where(cols == i1, -1e30, sc)
    m2 = jnp.max(sc2, axis=-1, keepdims=True)
    i2 = jnp.min(jnp.where(sc2 >= m2, cols, E), axis=-1, keepdims=True)

    w1 = jnp.sum(jnp.where(cols == i1, s, 0.0), axis=-1, keepdims=True)
    w2 = jnp.sum(jnp.where(cols == i2, s, 0.0), axis=-1, keepdims=True)
    den = w1 + w2 + 1e-20
    w1 = w1 / den
    w2 = w2 / den

    oh1 = (cols == i1).astype(jnp.float32)
    oh2 = (cols == i2).astype(jnp.float32)
    c1x = _cumsum0(oh1) - oh1
    c2x = _cumsum0(oh2) - oh2
    rank1 = jnp.sum((c1x + c2x) * oh1, axis=-1, keepdims=True)
    rank2 = jnp.sum((c1x + c2x + oh1) * oh2, axis=-1, keepdims=True)
    counts = jnp.sum(oh1 + oh2, axis=0, keepdims=True)

    nblk = jnp.ceil(counts / BLK)
    tri = (lax.broadcasted_iota(jnp.int32, (128, 128), 0)
           <= lax.broadcasted_iota(jnp.int32, (128, 128), 1)
           ).astype(jnp.float32)
    cblk = jnp.dot(nblk, tri, preferred_element_type=jnp.float32)
    start_blk = cblk - nblk
    aligned = start_blk * BLK
    dest1 = jnp.sum(aligned * oh1, axis=-1, keepdims=True) + rank1
    dest2 = jnp.sum(aligned * oh2, axis=-1, keepdims=True) + rank2

    is0 = (cols == 0).astype(jnp.float32)
    is1 = (cols == 1).astype(jnp.float32)
    dest_ref[...] = (dest1 * is0 + dest2 * is1).astype(jnp.int32)
    wflat_ref[...] = w1 * is0 + w2 * is1

    brow = lax.broadcasted_iota(jnp.int32, (128, 128), 0).astype(jnp.float32)
    used = lax.broadcasted_iota(jnp.int32, (128, 128), 1) < E
    ge = jnp.where(used, (brow >= cblk).astype(jnp.int32), 0)
    gid = jnp.minimum(jnp.sum(ge, axis=-1, keepdims=True), E - 1)
    gid_ref[...] = jnp.broadcast_to(gid, (128, 128)).astype(jnp.int32)


def _router_plan(logits, e_bias_pad):
    return pl.pallas_call(
        _plan_body,
        grid=(1,),
        in_specs=[
            pl.BlockSpec((T, 128), lambda i: (0, 0)),
            pl.BlockSpec((1, 128), lambda i: (0, 0)),
        ],
        out_specs=[
            pl.BlockSpec((T, 128), lambda i: (0, 0)),
            pl.BlockSpec((T, 128), lambda i: (0, 0)),
            pl.BlockSpec((128, 128), lambda i: (0, 0)),
        ],
        out_shape=[
            jax.ShapeDtypeStruct((T, 128), jnp.int32),
            jax.ShapeDtypeStruct((T, 128), jnp.float32),
            jax.ShapeDtypeStruct((128, 128), jnp.int32),
        ],
    )(logits, e_bias_pad)


def _gffn_body(gid_ref, xg_ref, wg_ref, wu_ref, wd_ref, o_ref):
    x = xg_ref[...]
    g = jnp.dot(x, wg_ref[0].T, preferred_element_type=jnp.float32)
    u = jnp.dot(x, wu_ref[0].T, preferred_element_type=jnp.float32)
    act = (g / (1.0 + jnp.exp(-g))) * u
    o_ref[...] = jnp.dot(act, wd_ref[0], preferred_element_type=jnp.float32)


def _grouped_ffn(xg, w_gate, w_up, w_down, gid):
    grid_spec = pltpu.PrefetchScalarGridSpec(
        num_scalar_prefetch=1,
        grid=(NBLOCKS,),
        in_specs=[
            pl.BlockSpec((BLK, H), lambda b, gid: (b, 0)),
            pl.BlockSpec((1, FF, H), lambda b, gid: (gid[b], 0, 0)),
            pl.BlockSpec((1, FF, H), lambda b, gid: (gid[b], 0, 0)),
            pl.BlockSpec((1, FF, H), lambda b, gid: (gid[b], 0, 0)),
        ],
        out_specs=pl.BlockSpec((BLK, H), lambda b, gid: (b, 0)),
    )
    return pl.pallas_call(
        _gffn_body,
        grid_spec=grid_spec,
        out_shape=jax.ShapeDtypeStruct((P, H), jnp.float32),
    )(gid, xg, w_gate, w_up, w_down)


def _shared_body(x_ref, wg_ref, wu_ref, wd_ref, base_ref, o_ref, acc):
    f = pl.program_id(1)
    nf = pl.num_programs(1)

    @pl.when(f == 0)
    def _():
        acc[...] = jnp.zeros_like(acc)

    x = x_ref[...]
    g = jnp.dot(x, wg_ref[...].T, preferred_element_type=jnp.float32)
    u = jnp.dot(x, wu_ref[...].T, preferred_element_type=jnp.float32)
    act = (g / (1.0 + jnp.exp(-g))) * u
    acc[...] += jnp.dot(act, wd_ref[...].T,
                        preferred_element_type=jnp.float32)

    @pl.when(f == nf - 1)
    def _():
        o_ref[...] = acc[...] + base_ref[...]


def _shared_ffn(h2, sh_gate, sh_up, sh_down, base):
    BT, FT = 256, 512
    return pl.pallas_call(
        _shared_body,
        grid=(T // BT, FF // FT),
        in_specs=[
            pl.BlockSpec((BT, H), lambda i, f: (i, 0)),
            pl.BlockSpec((FT, H), lambda i, f: (f, 0)),
            pl.BlockSpec((FT, H), lambda i, f: (f, 0)),
            pl.BlockSpec((H, FT), lambda i, f: (0, f)),
            pl.BlockSpec((BT, H), lambda i, f: (i, 0)),
        ],
        out_specs=pl.BlockSpec((BT, H), lambda i, f: (i, 0)),
        out_shape=jax.ShapeDtypeStruct((T, H), jnp.float32),
        scratch_shapes=[pltpu.VMEM((BT, H), jnp.float32)],
    )(h2, sh_gate, sh_up, sh_down, base)


def kernel(positions, hidden_states, input_ln_w, qkv_w, q_norm_w, k_norm_w,
           o_w, post_ln_w, gate_w, e_bias, w_gate_up, w_down,
           sh_gate_up, sh_down):
    x = hidden_states
    qkv = _qkv_proj(x, qkv_w, input_ln_w)
    inv = 1.0 / (10000.0 ** (jnp.arange(0, ROT, 2, dtype=jnp.float32) / ROT))
    ang = positions.astype(jnp.float32)[:, None] * inv[None, :]
    attn = _attention(qkv, jnp.cos(ang), jnp.sin(ang), q_norm_w, k_norm_w)

    gate_w_pad = jnp.zeros((128, H), jnp.float32).at[:E].set(gate_w)
    r2, h2, logits = _oproj(attn, o_w, x, post_ln_w, gate_w_pad)

    bias_pad = jnp.full((1, 128), -1e30, jnp.float32).at[0, :E].set(e_bias)
    dest2d, wflat2d, gid2d = _router_plan(logits, bias_pad)
    pos2 = dest2d[:, :TOPK]
    w2 = wflat2d[:, :TOPK]
    dest = pos2.reshape(NASSIGN)
    gid = gid2d[:NBLOCKS, 0]

    tok = jnp.arange(NASSIGN, dtype=jnp.int32) // TOPK
    xg = jnp.zeros((P, H), jnp.float32).at[dest].set(h2[tok])

    w_gate = w_gate_up[:, :FF, :]
    w_up = w_gate_up[:, FF:, :]
    out_pad = _grouped_ffn(xg, w_gate, w_up, w_down, gid)

    sh_gate = sh_gate_up[:FF, :]
    sh_up = sh_gate_up[FF:, :]
    base = _shared_ffn(h2, sh_gate, sh_up, sh_down, r2)

    routed = (out_pad[pos2[:, 0]] * w2[:, 0:1]
              + out_pad[pos2[:, 1]] * w2[:, 1:2])
    return base + routed * RSF


# speedup vs baseline: 1.4230x; 1.0616x over previous
import functools
import jax
import jax.numpy as jnp
from jax import lax
from jax.experimental import pallas as pl
from jax.experimental.pallas import tpu as pltpu
from jax.experimental.pallas import tpu_sc as plsc

H = 2048
NH = 16
NKV = 4
HD = 128
ROT = 64
E = 16
TOPK = 2
FF = 1024
EPS = 1e-05
RSF = 1.0
T = 2048

QKV = NH * HD + 2 * NKV * HD

BLK = 128
NASSIGN = T * TOPK
P = NASSIGN + E * BLK
NBLOCKS = P // BLK


def _qkv_body(x_ref, w_ref, ln_ref, o_ref):
    x = x_ref[...]
    v = jnp.mean(jnp.square(x), axis=-1, keepdims=True)
    h = x * lax.rsqrt(v + EPS) * ln_ref[...]
    o_ref[...] = jnp.dot(h, w_ref[...].T, preferred_element_type=jnp.float32)


def _qkv_proj(x, qkv_w, ln_w):
    BT, BN = 256, 512
    return pl.pallas_call(
        _qkv_body,
        grid=(T // BT, QKV // BN),
        in_specs=[
            pl.BlockSpec((BT, H), lambda i, j: (i, 0)),
            pl.BlockSpec((BN, H), lambda i, j: (j, 0)),
            pl.BlockSpec((1, H), lambda i, j: (0, 0)),
        ],
        out_specs=pl.BlockSpec((BT, BN), lambda i, j: (i, j)),
        out_shape=jax.ShapeDtypeStruct((T, QKV), jnp.float32),
    )(x, qkv_w, ln_w.reshape(1, H))


def _rope(x, cos, sin):
    x1 = x[:, : ROT // 2]
    x2 = x[:, ROT // 2: ROT]
    xp = x[:, ROT:]
    return jnp.concatenate([x1 * cos - x2 * sin, x2 * cos + x1 * sin, xp],
                           axis=-1)


def _rms(x, w):
    v = jnp.mean(jnp.square(x), axis=-1, keepdims=True)
    return x * lax.rsqrt(v + EPS) * w


BQA = 512


def _attn_body(qkv_ref, cos_ref, sin_ref, qn_ref, kn_ref, o_ref, smat):
    h = pl.program_id(0)
    qb = pl.program_id(1)
    g = h // (NH // NKV)
    q = qkv_ref[pl.ds(qb * BQA, BQA), pl.ds(h * HD, HD)]
    qcos = cos_ref[pl.ds(qb * BQA, BQA), :]
    qsin = sin_ref[pl.ds(qb * BQA, BQA), :]
    q = _rope(_rms(q, qn_ref[...]), qcos, qsin)

    for j in range(T // BQA):
        @pl.when(j <= qb)
        def _():
            k = qkv_ref[pl.ds(j * BQA, BQA), pl.ds(NH * HD + g * HD, HD)]
            k = _rope(_rms(k, kn_ref[...]),
                      cos_ref[pl.ds(j * BQA, BQA), :],
                      sin_ref[pl.ds(j * BQA, BQA), :])
            s = jnp.dot(q, k.T, preferred_element_type=jnp.float32)
            s = s * (HD ** -0.5)
            @pl.when(j == qb)
            def _():
                rows = lax.broadcasted_iota(jnp.int32, (BQA, BQA), 0)
                cols = lax.broadcasted_iota(jnp.int32, (BQA, BQA), 1)
                smat[:, pl.ds(j * BQA, BQA)] = jnp.where(
                    cols <= rows, s, -1e30)
            @pl.when(j < qb)
            def _():
                smat[:, pl.ds(j * BQA, BQA)] = s
        @pl.when(j > qb)
        def _():
            smat[:, pl.ds(j * BQA, BQA)] = jnp.full((BQA, BQA), -1e30,
                                                    jnp.float32)

    s = smat[...]
    m = jnp.max(s, axis=-1, keepdims=True)
    p = jnp.exp(s - m)
    p = p / jnp.sum(p, axis=-1, keepdims=True)
    v = qkv_ref[:, pl.ds(NH * HD + NKV * HD + g * HD, HD)]
    o_ref[...] = jnp.dot(p, v, preferred_element_type=jnp.float32)


def _attention(qkv, cos, sin, qn, kn):
    return pl.pallas_call(
        _attn_body,
        grid=(NH, T // BQA),
        in_specs=[
            pl.BlockSpec((T, QKV), lambda h, i: (0, 0)),
            pl.BlockSpec((T, ROT // 2), lambda h, i: (0, 0)),
            pl.BlockSpec((T, ROT // 2), lambda h, i: (0, 0)),
            pl.BlockSpec((1, HD), lambda h, i: (0, 0)),
            pl.BlockSpec((1, HD), lambda h, i: (0, 0)),
        ],
        out_specs=pl.BlockSpec((BQA, HD), lambda h, i: (i, h)),
        out_shape=jax.ShapeDtypeStruct((T, NH * HD), jnp.float32),
        scratch_shapes=[
            pltpu.VMEM((BQA, T), jnp.float32),
        ],
    )(qkv, cos, sin, qn.reshape(1, HD), kn.reshape(1, HD))


def _oproj_body(a_ref, w_ref, res_ref, ln_ref, gw_ref,
                r2_ref, h2_ref, lg_ref, acc):
    kk = pl.program_id(1)
    nk = pl.num_programs(1)

    @pl.when(kk == 0)
    def _():
        acc[...] = jnp.zeros_like(acc)

    acc[...] += jnp.dot(a_ref[...], w_ref[...].T,
                        preferred_element_type=jnp.float32)

    @pl.when(kk == nk - 1)
    def _():
        r2 = acc[...] + res_ref[...]
        r2_ref[...] = r2
        v = jnp.mean(jnp.square(r2), axis=-1, keepdims=True)
        h2 = r2 * lax.rsqrt(v + EPS) * ln_ref[...]
        h2_ref[...] = h2
        lg_ref[...] = jnp.dot(h2, gw_ref[...].T,
                              preferred_element_type=jnp.float32)


def _oproj(attn, o_w, residual, post_ln_w, gate_w_pad):
    BT, BK = 256, 512
    return pl.pallas_call(
        _oproj_body,
        grid=(T // BT, H // BK),
        in_specs=[
            pl.BlockSpec((BT, BK), lambda i, k: (i, k)),
            pl.BlockSpec((H, BK), lambda i, k: (0, k)),
            pl.BlockSpec((BT, H), lambda i, k: (i, 0)),
            pl.BlockSpec((1, H), lambda i, k: (0, 0)),
            pl.BlockSpec((128, H), lambda i, k: (0, 0)),
        ],
        out_specs=[
            pl.BlockSpec((BT, H), lambda i, k: (i, 0)),
            pl.BlockSpec((BT, H), lambda i, k: (i, 0)),
            pl.BlockSpec((BT, 128), lambda i, k: (i, 0)),
        ],
        out_shape=[
            jax.ShapeDtypeStruct((T, H), jnp.float32),
            jax.ShapeDtypeStruct((T, H), jnp.float32),
            jax.ShapeDtypeStruct((T, 128), jnp.float32),
        ],
        scratch_shapes=[pltpu.VMEM((BT, H), jnp.float32)],
    )(attn, o_w, residual, post_ln_w.reshape(1, H), gate_w_pad)


def _cumsum0(y):
    n = y.shape[0]
    k = 1
    while k < n:
        y = y + jnp.concatenate(
            [jnp.zeros((k, y.shape[1]), y.dtype), y[:-k]], axis=0)
        k *= 2
    return y


def _plan_body(lg_ref, bias_ref, dest_ref, wflat_ref, gid_ref):
    lg = lg_ref[...]
    cols = lax.broadcasted_iota(jnp.int32, (T, 128), 1)
    valid = cols < E
    s = jnp.where(valid, 1.0 / (1.0 + jnp.exp(-lg)), 0.0)
    sc = jnp.where(valid, s + bias_ref[...], -1e30)

    m1 = jnp.max(sc, axis=-1, keepdims=True)
    i1 = jnp.min(jnp.where(sc >= m1, cols, E), axis=-1, keepdims=True)
    sc2 = jnp.where(cols == i1, -1e30, sc)
    m2 = jnp.max(sc2, axis=-1, keepdims=True)
    i2 = jnp.min(jnp.where(sc2 >= m2, cols, E), axis=-1, keepdims=True)

    w1 = jnp.sum(jnp.where(cols == i1, s, 0.0), axis=-1, keepdims=True)
    w2 = jnp.sum(jnp.where(cols == i2, s, 0.0), axis=-1, keepdims=True)
    den = w1 + w2 + 1e-20
    w1 = w1 / den
    w2 = w2 / den

    oh1 = (cols == i1).astype(jnp.float32)
    oh2 = (cols == i2).astype(jnp.float32)
    c1x = _cumsum0(oh1) - oh1
    c2x = _cumsum0(oh2) - oh2
    rank1 = jnp.sum((c1x + c2x) * oh1, axis=-1, keepdims=True)
    rank2 = jnp.sum((c1x + c2x + oh1) * oh2, axis=-1, keepdims=True)
    counts = jnp.sum(oh1 + oh2, axis=0, keepdims=True)

    nblk = jnp.ceil(counts / BLK)
    tri = (lax.broadcasted_iota(jnp.int32, (128, 128), 0)
           <= lax.broadcasted_iota(jnp.int32, (128, 128), 1)
           ).astype(jnp.float32)
    cblk = jnp.dot(nblk, tri, preferred_element_type=jnp.float32)
    start_blk = cblk - nblk
    aligned = start_blk * BLK
    dest1 = jnp.sum(aligned * oh1, axis=-1, keepdims=True) + rank1
    dest2 = jnp.sum(aligned * oh2, axis=-1, keepdims=True) + rank2

    is0 = (cols == 0).astype(jnp.float32)
    is1 = (cols == 1).astype(jnp.float32)
    dest_ref[...] = (dest1 * is0 + dest2 * is1).astype(jnp.int32)
    wflat_ref[...] = w1 * is0 + w2 * is1

    brow = lax.broadcasted_iota(jnp.int32, (128, 128), 0).astype(jnp.float32)
    used = lax.broadcasted_iota(jnp.int32, (128, 128), 1) < E
    ge = jnp.where(used, (brow >= cblk).astype(jnp.int32), 0)
    gid = jnp.minimum(jnp.sum(ge, axis=-1, keepdims=True), E - 1)
    gid_ref[...] = jnp.broadcast_to(gid, (128, 128)).astype(jnp.int32)


def _router_plan(logits, e_bias_pad):
    return pl.pallas_call(
        _plan_body,
        grid=(1,),
        in_specs=[
            pl.BlockSpec((T, 128), lambda i: (0, 0)),
            pl.BlockSpec((1, 128), lambda i: (0, 0)),
        ],
        out_specs=[
            pl.BlockSpec((T, 128), lambda i: (0, 0)),
            pl.BlockSpec((T, 128), lambda i: (0, 0)),
            pl.BlockSpec((128, 128), lambda i: (0, 0)),
        ],
        out_shape=[
            jax.ShapeDtypeStruct((T, 128), jnp.int32),
            jax.ShapeDtypeStruct((T, 128), jnp.float32),
            jax.ShapeDtypeStruct((128, 128), jnp.int32),
        ],
    )(logits, e_bias_pad)


def _gffn_body(gid_ref, xg_ref, wg_ref, wu_ref, wd_ref, o_ref):
    x = xg_ref[...]
    g = jnp.dot(x, wg_ref[0].T, preferred_element_type=jnp.float32)
    u = jnp.dot(x, wu_ref[0].T, preferred_element_type=jnp.float32)
    act = (g / (1.0 + jnp.exp(-g))) * u
    o_ref[...] = jnp.dot(act, wd_ref[0], preferred_element_type=jnp.float32)


def _grouped_ffn(xg, w_gate, w_up, w_down, gid):
    grid_spec = pltpu.PrefetchScalarGridSpec(
        num_scalar_prefetch=1,
        grid=(NBLOCKS,),
        in_specs=[
            pl.BlockSpec((BLK, H), lambda b, gid: (b, 0)),
            pl.BlockSpec((1, FF, H), lambda b, gid: (gid[b], 0, 0)),
            pl.BlockSpec((1, FF, H), lambda b, gid: (gid[b], 0, 0)),
            pl.BlockSpec((1, FF, H), lambda b, gid: (gid[b], 0, 0)),
        ],
        out_specs=pl.BlockSpec((BLK, H), lambda b, gid: (b, 0)),
    )
    return pl.pallas_call(
        _gffn_body,
        grid_spec=grid_spec,
        out_shape=jax.ShapeDtypeStruct((P, H), jnp.float32),
    )(gid, xg, w_gate, w_up, w_down)


def _shared_body(x_ref, wg_ref, wu_ref, wd_ref, base_ref, o_ref, acc):
    f = pl.program_id(1)
    nf = pl.num_programs(1)

    @pl.when(f == 0)
    def _():
        acc[...] = jnp.zeros_like(acc)

    x = x_ref[...]
    g = jnp.dot(x, wg_ref[...].T, preferred_element_type=jnp.float32)
    u = jnp.dot(x, wu_ref[...].T, preferred_element_type=jnp.float32)
    act = (g / (1.0 + jnp.exp(-g))) * u
    acc[...] += jnp.dot(act, wd_ref[...].T,
                        preferred_element_type=jnp.float32)

    @pl.when(f == nf - 1)
    def _():
        o_ref[...] = acc[...] + base_ref[...]


def _shared_ffn(h2, sh_gate, sh_up, sh_down, base):
    BT, FT = 256, 512
    return pl.pallas_call(
        _shared_body,
        grid=(T // BT, FF // FT),
        in_specs=[
            pl.BlockSpec((BT, H), lambda i, f: (i, 0)),
            pl.BlockSpec((FT, H), lambda i, f: (f, 0)),
            pl.BlockSpec((FT, H), lambda i, f: (f, 0)),
            pl.BlockSpec((H, FT), lambda i, f: (0, f)),
            pl.BlockSpec((BT, H), lambda i, f: (i, 0)),
        ],
        out_specs=pl.BlockSpec((BT, H), lambda i, f: (i, 0)),
        out_shape=jax.ShapeDtypeStruct((T, H), jnp.float32),
        scratch_shapes=[pltpu.VMEM((BT, H), jnp.float32)],
    )(h2, sh_gate, sh_up, sh_down, base)


_NC = 2
_NS = 16
_NW = _NC * _NS
_LANES = 16


def _dispatch_body(h2_hbm, dest_hbm, xg_hbm, tok_v, dest_v, rows_v, sem):
    wid = lax.axis_index("s") * _NC + lax.axis_index("c")
    per_w = NASSIGN // _NW
    csz = 32
    pat = lax.shift_right_arithmetic(
        lax.broadcasted_iota(jnp.int32, (_LANES,), 0), 1)
    for c in range(per_w // csz):
        base = wid * per_w + c * csz
        for half in range(csz // _LANES):
            tok_v[pl.ds(half * _LANES, _LANES)] = (
                pat + (base + half * _LANES) // TOPK)
        pltpu.sync_copy(dest_hbm.at[pl.ds(base, csz)], dest_v)
        pltpu.async_copy(h2_hbm.at[tok_v], rows_v, sem).wait()
        pltpu.async_copy(rows_v, xg_hbm.at[dest_v], sem).wait()


def _sc_dispatch(h2, dest):
    mesh = plsc.VectorSubcoreMesh(core_axis_name="c", subcore_axis_name="s")
    k = functools.partial(
        pl.kernel,
        mesh=mesh,
        out_type=jax.ShapeDtypeStruct((P, H), jnp.float32),
        scratch_types=[
            pltpu.VMEM((32,), jnp.int32),
            pltpu.VMEM((32,), jnp.int32),
            pltpu.VMEM((32, H), jnp.float32),
            pltpu.SemaphoreType.DMA,
        ],
    )(_dispatch_body)
    return k(h2, dest)


def _gather2_body(outpad_hbm, i0_hbm, i1_hbm, r0_hbm, r1_hbm,
                  i0_v, i1_v, r0_v, r1_v, sem0, sem1):
    wid = lax.axis_index("s") * _NC + lax.axis_index("c")
    per_w = T // _NW
    csz = 16
    for c in range(per_w // csz):
        tbase = wid * per_w + c * csz
        pltpu.sync_copy(i0_hbm.at[pl.ds(tbase, csz)], i0_v)
        pltpu.sync_copy(i1_hbm.at[pl.ds(tbase, csz)], i1_v)
        cp0 = pltpu.async_copy(outpad_hbm.at[i0_v], r0_v, sem0)
        cp1 = pltpu.async_copy(outpad_hbm.at[i1_v], r1_v, sem1)
        cp0.wait()
        cp1.wait()
        pltpu.sync_copy(r0_v, r0_hbm.at[pl.ds(tbase, csz)])
        pltpu.sync_copy(r1_v, r1_hbm.at[pl.ds(tbase, csz)])


def _sc_gather2(out_pad, pos0, pos1):
    mesh = plsc.VectorSubcoreMesh(core_axis_name="c", subcore_axis_name="s")
    k = functools.partial(
        pl.kernel,
        mesh=mesh,
        out_type=[
            jax.ShapeDtypeStruct((T, H), jnp.float32),
            jax.ShapeDtypeStruct((T, H), jnp.float32),
        ],
        scratch_types=[
            pltpu.VMEM((16,), jnp.int32),
            pltpu.VMEM((16,), jnp.int32),
            pltpu.VMEM((16, H), jnp.float32),
            pltpu.VMEM((16, H), jnp.float32),
            pltpu.SemaphoreType.DMA,
            pltpu.SemaphoreType.DMA,
        ],
    )(_gather2_body)
    return k(out_pad, pos0, pos1)


def _fin_body(base_ref, r0_ref, r1_ref, wf_ref, o_ref):
    w0 = wf_ref[:, 0:1]
    w1 = wf_ref[:, 1:2]
    o_ref[...] = (base_ref[...]
                  + (w0 * r0_ref[...] + w1 * r1_ref[...]) * RSF)


def _fin_combine(base, r0, r1, wf):
    BT = 256
    return pl.pallas_call(
        _fin_body,
        grid=(T // BT,),
        in_specs=[
            pl.BlockSpec((BT, H), lambda i: (i, 0)),
            pl.BlockSpec((BT, H), lambda i: (i, 0)),
            pl.BlockSpec((BT, H), lambda i: (i, 0)),
            pl.BlockSpec((BT, 128), lambda i: (i, 0)),
        ],
        out_specs=pl.BlockSpec((BT, H), lambda i: (i, 0)),
        out_shape=jax.ShapeDtypeStruct((T, H), jnp.float32),
    )(base, r0, r1, wf)


def kernel(positions, hidden_states, input_ln_w, qkv_w, q_norm_w, k_norm_w,
           o_w, post_ln_w, gate_w, e_bias, w_gate_up, w_down,
           sh_gate_up, sh_down):
    x = hidden_states
    qkv = _qkv_proj(x, qkv_w, input_ln_w)
    inv = 1.0 / (10000.0 ** (jnp.arange(0, ROT, 2, dtype=jnp.float32) / ROT))
    ang = positions.astype(jnp.float32)[:, None] * inv[None, :]
    attn = _attention(qkv, jnp.cos(ang), jnp.sin(ang), q_norm_w, k_norm_w)

    gate_w_pad = jnp.zeros((128, H), jnp.float32).at[:E].set(gate_w)
    r2, h2, logits = _oproj(attn, o_w, x, post_ln_w, gate_w_pad)

    bias_pad = jnp.full((1, 128), -1e30, jnp.float32).at[0, :E].set(e_bias)
    dest2d, wflat2d, gid2d = _router_plan(logits, bias_pad)
    pos2 = dest2d[:, :TOPK]
    w2 = wflat2d[:, :TOPK]
    dest = pos2.reshape(NASSIGN)
    gid = gid2d[:NBLOCKS, 0]

    xg = _sc_dispatch(h2, dest)

    w_gate = w_gate_up[:, :FF, :]
    w_up = w_gate_up[:, FF:, :]
    out_pad = _grouped_ffn(xg, w_gate, w_up, w_down, gid)

    sh_gate = sh_gate_up[:FF, :]
    sh_up = sh_gate_up[FF:, :]
    base = _shared_ffn(h2, sh_gate, sh_up, sh_down, r2)

    r0, r1 = _sc_gather2(out_pad, pos2[:, 0], pos2[:, 1])
    return _fin_combine(base, r0, r1, wflat2d)
